# Initial kernel scaffold; baseline (speedup 1.0000x reference)
import jax
import jax.numpy as jnp
from jax.experimental import pallas as pl


def kernel(x, edge_index, W0a, b0a, g0a, be0a, W1a, b1a, g_bn1, b_bn1, W0b, b0b, g0b, be0b, W1b, b1b):
    raise NotImplementedError("write your pallas kernel here")


# speedup vs baseline: 4.4929x; 4.4929x over previous
import functools

import jax
import jax.numpy as jnp
from jax import lax
from jax.experimental import pallas as pl
from jax.experimental.pallas import tpu as pltpu
from jax.experimental.pallas import tpu_sc as plsc

_N = 10000
_E = 320000
_DIN = 128
_H = 64
_DOUT = 128

_NC = 2
_NS = 16
_NW = _NC * _NS
_CHUNK = 128
_KCH = 80
_EPAD = _NW * _KCH * _CHUNK
_NPAD = 10240


def _segsum_body(tbl_hbm, srcm_hbm, dstm_hbm, zeros_hbm, out_hbm,
                 src_v, dst_v, row_v, acc_sh, sem):
    c = lax.axis_index("c")
    s = lax.axis_index("s")
    wid = s * _NC + c
    rows_per_sub = _NPAD // _NS
    r0 = s * rows_per_sub

    for i in range(rows_per_sub // _CHUNK):
        pltpu.sync_copy(zeros_hbm, acc_sh.at[pl.ds(r0 + i * _CHUNK, _CHUNK)])
    plsc.subcore_barrier()

    pltpu.sync_copy(srcm_hbm.at[pl.ds(wid * _KCH, _KCH)], src_v)
    pltpu.sync_copy(dstm_hbm.at[pl.ds(wid * _KCH, _KCH)], dst_v)

    def body(j, carry):
        pltpu.async_copy(tbl_hbm.at[src_v.at[j]], row_v, sem).wait()
        pltpu.sync_copy(row_v, acc_sh.at[dst_v.at[j]], add=True)
        return carry

    lax.fori_loop(0, _KCH, body, 0)
    plsc.subcore_barrier()

    pltpu.sync_copy(acc_sh.at[pl.ds(r0, rows_per_sub)],
                    out_hbm.at[pl.ds(c * _NPAD + r0, rows_per_sub)])


_segsum = pl.kernel(
    _segsum_body,
    mesh=plsc.VectorSubcoreMesh(core_axis_name="c", subcore_axis_name="s"),
    out_type=jax.ShapeDtypeStruct((_NC * _NPAD, _H), jnp.float32),
    scratch_types=[
        pltpu.VMEM((_KCH, _CHUNK), jnp.int32),
        pltpu.VMEM((_KCH, _CHUNK), jnp.int32),
        pltpu.VMEM((_CHUNK, _H), jnp.float32),
        pltpu.VMEM_SHARED((_NPAD, _H), jnp.float32),
        pltpu.SemaphoreType.DMA,
    ],
    compiler_params=pltpu.CompilerParams(use_tc_tiling_on_sc=False),
)


def _bn(t, g, b):
    mean = jnp.mean(t, axis=0, keepdims=True)
    var = jnp.mean(jnp.square(t - mean), axis=0, keepdims=True)
    return (t - mean) * lax.rsqrt(var + 1e-5) * g + b


def _mm_k(x_ref, w_ref, o_ref):
    o_ref[...] = jnp.dot(x_ref[...], w_ref[...],
                         preferred_element_type=jnp.float32)


def _stage_b_k(p_ref, part_ref, b0a_ref, g0a_ref, be0a_ref, w1a_ref, b1a_ref,
               gbn_ref, bbn_ref, w0b_ref, z_ref, q_ref):
    agg = part_ref[0, :, :] + part_ref[1, :, :]
    t = p_ref[...] + agg[:_N] + b0a_ref[...]
    y = jnp.maximum(_bn(t, g0a_ref[...], be0a_ref[...]), 0.0)
    z = jnp.dot(y, w1a_ref[...], preferred_element_type=jnp.float32) + b1a_ref[...]
    z_ref[...] = z
    hh = jnp.maximum(_bn(z, gbn_ref[...], bbn_ref[...]), 0.0)
    q_ref[...] = jnp.dot(hh, w0b_ref[...], preferred_element_type=jnp.float32)


def _stage_c_k(q_ref, part_ref, b0b_ref, g0b_ref, be0b_ref, w1b_ref, b1b_ref,
               o_ref):
    agg = part_ref[0, :, :] + part_ref[1, :, :]
    t = q_ref[...] + agg[:_N] + b0b_ref[...]
    y = jnp.maximum(_bn(t, g0b_ref[...], be0b_ref[...]), 0.0)
    o_ref[...] = jnp.dot(y, w1b_ref[...],
                         preferred_element_type=jnp.float32) + b1b_ref[...]


def kernel(x, edge_index, W0a, b0a, g0a, be0a, W1a, b1a, g_bn1, b_bn1,
           W0b, b0b, g0b, be0b, W1b, b1b):
    src = edge_index[0]
    dst = edge_index[1]
    npe = _EPAD - _E
    srcp = jnp.concatenate(
        [src, jnp.zeros((npe,), jnp.int32)]).reshape(_NW * _KCH, _CHUNK)
    dstp = jnp.concatenate(
        [dst, _N + (jnp.arange(npe, dtype=jnp.int32) % (_NPAD - _N))]
    ).reshape(_NW * _KCH, _CHUNK)
    zeros128 = jnp.zeros((_CHUNK, _H), jnp.float32)

    p1 = pl.pallas_call(
        _mm_k, out_shape=jax.ShapeDtypeStruct((_N, _H), jnp.float32))(x, W0a)

    part1 = _segsum(p1, srcp, dstp, zeros128).reshape(_NC, _NPAD, _H)

    z, q = pl.pallas_call(
        _stage_b_k,
        out_shape=(jax.ShapeDtypeStruct((_N, _H), jnp.float32),
                   jax.ShapeDtypeStruct((_N, _H), jnp.float32)),
    )(p1, part1, b0a.reshape(1, _H), g0a.reshape(1, _H), be0a.reshape(1, _H),
      W1a, b1a.reshape(1, _H), g_bn1.reshape(1, _H), b_bn1.reshape(1, _H),
      W0b)

    part2 = _segsum(q, srcp, dstp, zeros128).reshape(_NC, _NPAD, _H)

    out = pl.pallas_call(
        _stage_c_k,
        out_shape=jax.ShapeDtypeStruct((_N, _DOUT), jnp.float32),
    )(q, part2, b0b.reshape(1, _H), g0b.reshape(1, _H), be0b.reshape(1, _H),
      W1b, b1b.reshape(1, _DOUT))

    return (out, z)


# speedup vs baseline: 11.2324x; 2.5000x over previous
import functools

import jax
import jax.numpy as jnp
from jax import lax
from jax.experimental import pallas as pl
from jax.experimental.pallas import tpu as pltpu
from jax.experimental.pallas import tpu_sc as plsc

_N = 10000
_E = 320000
_DIN = 128
_H = 64
_DOUT = 128

_NC = 2
_NS = 16
_NW = _NC * _NS
_CHUNK = 128
_KCH = 80
_EPAD = _NW * _KCH * _CHUNK
_NPAD = 10240


_NBUF = 2


def _segsum_body(tbl_hbm, srcm_hbm, dstm_hbm, zeros_hbm, out_hbm,
                 src_v, dst_v, row0, row1, tbl_sh, acc_sh,
                 sem0, sem1):
    rows = (row0, row1)
    sems = (sem0, sem1)
    c = lax.axis_index("c")
    s = lax.axis_index("s")
    wid = s * _NC + c
    rows_per_sub = _NPAD // _NS
    r0 = s * rows_per_sub

    for i in range(rows_per_sub // _CHUNK):
        pltpu.sync_copy(zeros_hbm, acc_sh.at[pl.ds(r0 + i * _CHUNK, _CHUNK)])
    tn = _N // _NS
    pltpu.sync_copy(tbl_hbm.at[pl.ds(s * tn, tn)], tbl_sh.at[pl.ds(s * tn, tn)])
    pltpu.sync_copy(srcm_hbm.at[pl.ds(wid * _KCH, _KCH)], src_v)
    pltpu.sync_copy(dstm_hbm.at[pl.ds(wid * _KCH, _KCH)], dst_v)
    plsc.subcore_barrier()

    for b in range(_NBUF):
        pltpu.async_copy(tbl_sh.at[src_v.at[b]], rows[b], sems[b])

    def body(j0, carry):
        for b in range(_NBUF):
            j = j0 + b
            pltpu.make_async_copy(tbl_sh.at[src_v.at[j]], rows[b],
                                  sems[b]).wait()
            pltpu.sync_copy(rows[b], acc_sh.at[dst_v.at[j]], add=True)
            pltpu.async_copy(tbl_sh.at[src_v.at[j + _NBUF]], rows[b], sems[b])
        return carry

    lax.fori_loop(0, (_KCH - _NBUF) // _NBUF,
                  lambda i, cr: body(i * _NBUF, cr), 0)
    for b in range(_NBUF):
        j = _KCH - _NBUF + b
        pltpu.make_async_copy(tbl_sh.at[src_v.at[j]], rows[b], sems[b]).wait()
        pltpu.sync_copy(rows[b], acc_sh.at[dst_v.at[j]], add=True)

    plsc.subcore_barrier()
    pltpu.sync_copy(acc_sh.at[pl.ds(r0, rows_per_sub)],
                    out_hbm.at[pl.ds(c * _NPAD + r0, rows_per_sub)])


_segsum = pl.kernel(
    _segsum_body,
    mesh=plsc.VectorSubcoreMesh(core_axis_name="c", subcore_axis_name="s"),
    out_type=jax.ShapeDtypeStruct((_NC * _NPAD, _H), jnp.float32),
    scratch_types=[
        pltpu.VMEM((_KCH, _CHUNK), jnp.int32),
        pltpu.VMEM((_KCH, _CHUNK), jnp.int32),
        pltpu.VMEM((_CHUNK, _H), jnp.float32),
        pltpu.VMEM((_CHUNK, _H), jnp.float32),
        pltpu.VMEM_SHARED((_N, _H), jnp.float32),
        pltpu.VMEM_SHARED((_NPAD, _H), jnp.float32),
        pltpu.SemaphoreType.DMA,
        pltpu.SemaphoreType.DMA,
    ],
    compiler_params=pltpu.CompilerParams(use_tc_tiling_on_sc=False),
)


def _bn(t, g, b):
    mean = jnp.mean(t, axis=0, keepdims=True)
    var = jnp.mean(jnp.square(t - mean), axis=0, keepdims=True)
    return (t - mean) * lax.rsqrt(var + 1e-5) * g + b


def _mm_k(x_ref, w_ref, o_ref):
    o_ref[...] = jnp.dot(x_ref[...], w_ref[...],
                         preferred_element_type=jnp.float32)


def _stage_b_k(p_ref, part_ref, b0a_ref, g0a_ref, be0a_ref, w1a_ref, b1a_ref,
               gbn_ref, bbn_ref, w0b_ref, z_ref, q_ref):
    agg = part_ref[0, :, :] + part_ref[1, :, :]
    t = p_ref[...] + agg[:_N] + b0a_ref[...]
    y = jnp.maximum(_bn(t, g0a_ref[...], be0a_ref[...]), 0.0)
    z = jnp.dot(y, w1a_ref[...], preferred_element_type=jnp.float32) + b1a_ref[...]
    z_ref[...] = z
    hh = jnp.maximum(_bn(z, gbn_ref[...], bbn_ref[...]), 0.0)
    q_ref[...] = jnp.dot(hh, w0b_ref[...], preferred_element_type=jnp.float32)


def _stage_c_k(q_ref, part_ref, b0b_ref, g0b_ref, be0b_ref, w1b_ref, b1b_ref,
               o_ref):
    agg = part_ref[0, :, :] + part_ref[1, :, :]
    t = q_ref[...] + agg[:_N] + b0b_ref[...]
    y = jnp.maximum(_bn(t, g0b_ref[...], be0b_ref[...]), 0.0)
    o_ref[...] = jnp.dot(y, w1b_ref[...],
                         preferred_element_type=jnp.float32) + b1b_ref[...]


def kernel(x, edge_index, W0a, b0a, g0a, be0a, W1a, b1a, g_bn1, b_bn1,
           W0b, b0b, g0b, be0b, W1b, b1b):
    src = edge_index[0]
    dst = edge_index[1]
    npe = _EPAD - _E
    srcp = jnp.concatenate(
        [src, jnp.zeros((npe,), jnp.int32)]).reshape(_NW * _KCH, _CHUNK)
    dstp = jnp.concatenate(
        [dst, _N + (jnp.arange(npe, dtype=jnp.int32) % (_NPAD - _N))]
    ).reshape(_NW * _KCH, _CHUNK)
    zeros128 = jnp.zeros((_CHUNK, _H), jnp.float32)

    p1 = pl.pallas_call(
        _mm_k, out_shape=jax.ShapeDtypeStruct((_N, _H), jnp.float32))(x, W0a)

    part1 = _segsum(p1, srcp, dstp, zeros128).reshape(_NC, _NPAD, _H)

    z, q = pl.pallas_call(
        _stage_b_k,
        out_shape=(jax.ShapeDtypeStruct((_N, _H), jnp.float32),
                   jax.ShapeDtypeStruct((_N, _H), jnp.float32)),
    )(p1, part1, b0a.reshape(1, _H), g0a.reshape(1, _H), be0a.reshape(1, _H),
      W1a, b1a.reshape(1, _H), g_bn1.reshape(1, _H), b_bn1.reshape(1, _H),
      W0b)

    part2 = _segsum(q, srcp, dstp, zeros128).reshape(_NC, _NPAD, _H)

    out = pl.pallas_call(
        _stage_c_k,
        out_shape=jax.ShapeDtypeStruct((_N, _DOUT), jnp.float32),
    )(q, part2, b0b.reshape(1, _H), g0b.reshape(1, _H), be0b.reshape(1, _H),
      W1b, b1b.reshape(1, _DOUT))

    return (out, z)


# speedup vs baseline: 11.9700x; 1.0657x over previous
import functools

import jax
import jax.numpy as jnp
from jax import lax
from jax.experimental import pallas as pl
from jax.experimental.pallas import tpu as pltpu
from jax.experimental.pallas import tpu_sc as plsc

_N = 10000
_E = 320000
_DIN = 128
_H = 64
_DOUT = 128

_NC = 2
_NS = 16
_NW = _NC * _NS
_CHUNK = 128
_KCH = 80
_EPAD = _NW * _KCH * _CHUNK
_NPAD = 10240


_NBUF = 4
_NBI = 8


def _segsum_body(tbl_hbm, sdm_hbm, zeros_hbm, out_hbm,
                 rows, idxs, tbl_sh, acc_sh, gsems, ssems, isems):
    c = lax.axis_index("c")
    s = lax.axis_index("s")
    wid = s * _NC + c
    base = wid * _KCH
    rows_per_sub = _NPAD // _NS
    r0 = s * rows_per_sub

    for i in range(rows_per_sub // _CHUNK):
        pltpu.sync_copy(zeros_hbm, acc_sh.at[pl.ds(r0 + i * _CHUNK, _CHUNK)])
    tn = _N // _NS
    pltpu.sync_copy(tbl_hbm.at[pl.ds(s * tn, tn)], tbl_sh.at[pl.ds(s * tn, tn)])
    plsc.subcore_barrier()

    def iload_start(j, jb):
        pltpu.async_copy(sdm_hbm.at[base + j], idxs[jb], isems[jb])

    def iload_wait(j, jb):
        pltpu.make_async_copy(sdm_hbm.at[base + j], idxs[jb], isems[jb]).wait()

    def gather_start(jb, rb):
        pltpu.async_copy(tbl_sh.at[idxs[jb].at[0]], rows[rb], gsems[rb])

    def gather_wait(jb, rb):
        pltpu.make_async_copy(tbl_sh.at[idxs[jb].at[0]], rows[rb],
                              gsems[rb]).wait()

    def scat_start(jb, rb):
        pltpu.async_copy(rows[rb], acc_sh.at[idxs[jb].at[1]], ssems[rb],
                         add=True)

    def scat_wait(jb, rb):
        pltpu.make_async_copy(rows[rb], acc_sh.at[idxs[jb].at[1]],
                              ssems[rb]).wait()

    def step(j, b, guard_ssem=True, do_gather=True, do_iload=True):
        rb = b % _NBUF
        gather_wait(b, rb)
        scat_start(b, rb)
        if do_gather:
            bg = (b + 2) % _NBI
            rg = (b + 2) % _NBUF
            iload_wait(j + 2, bg)
            if guard_ssem:
                scat_wait((b - 2) % _NBI, rg)
            gather_start(bg, rg)
        if do_iload:
            iload_start(j + 4, (b + 4) % _NBI)

    for b in range(_NBUF):
        iload_start(b, b)
    for b in range(2):
        iload_wait(b, b)
        gather_start(b, b)
    for b in range(_NBI):
        step(b, b, guard_ssem=(b >= 2))

    def body(g, carry):
        j0 = g * _NBI
        for b in range(_NBI):
            step(j0 + b, b)
        return carry

    lax.fori_loop(1, (_KCH // _NBI) - 1, body, 0)

    for b in range(_NBI):
        j = _KCH - _NBI + b
        step(j, b, do_gather=(j + 2 < _KCH), do_iload=(j + 4 < _KCH))
    for b in range(_NBUF):
        scat_wait(_NBUF + b, b)

    plsc.subcore_barrier()
    pltpu.sync_copy(acc_sh.at[pl.ds(r0, rows_per_sub)],
                    out_hbm.at[pl.ds(c * _NPAD + r0, rows_per_sub)])


_segsum = pl.kernel(
    _segsum_body,
    mesh=plsc.VectorSubcoreMesh(core_axis_name="c", subcore_axis_name="s"),
    out_type=jax.ShapeDtypeStruct((_NC * _NPAD, _H), jnp.float32),
    scratch_types=[
        tuple(pltpu.VMEM((_CHUNK, _H), jnp.float32) for _ in range(_NBUF)),
        tuple(pltpu.VMEM((2, _CHUNK), jnp.int32) for _ in range(_NBI)),
        pltpu.VMEM_SHARED((_N, _H), jnp.float32),
        pltpu.VMEM_SHARED((_NPAD, _H), jnp.float32),
        tuple(pltpu.SemaphoreType.DMA for _ in range(_NBUF)),
        tuple(pltpu.SemaphoreType.DMA for _ in range(_NBUF)),
        tuple(pltpu.SemaphoreType.DMA for _ in range(_NBI)),
    ],
    compiler_params=pltpu.CompilerParams(use_tc_tiling_on_sc=False),
)


def _bn(t, g, b):
    mean = jnp.mean(t, axis=0, keepdims=True)
    var = jnp.mean(jnp.square(t - mean), axis=0, keepdims=True)
    return (t - mean) * lax.rsqrt(var + 1e-5) * g + b


def _mm_k(x_ref, w_ref, o_ref):
    o_ref[...] = jnp.dot(x_ref[...], w_ref[...],
                         preferred_element_type=jnp.float32)


def _stage_b_k(p_ref, part_ref, b0a_ref, g0a_ref, be0a_ref, w1a_ref, b1a_ref,
               gbn_ref, bbn_ref, w0b_ref, z_ref, q_ref):
    agg = part_ref[0, :, :] + part_ref[1, :, :]
    t = p_ref[...] + agg[:_N] + b0a_ref[...]
    y = jnp.maximum(_bn(t, g0a_ref[...], be0a_ref[...]), 0.0)
    z = jnp.dot(y, w1a_ref[...], preferred_element_type=jnp.float32) + b1a_ref[...]
    z_ref[...] = z
    hh = jnp.maximum(_bn(z, gbn_ref[...], bbn_ref[...]), 0.0)
    q_ref[...] = jnp.dot(hh, w0b_ref[...], preferred_element_type=jnp.float32)


def _stage_c_k(q_ref, part_ref, b0b_ref, g0b_ref, be0b_ref, w1b_ref, b1b_ref,
               o_ref):
    agg = part_ref[0, :, :] + part_ref[1, :, :]
    t = q_ref[...] + agg[:_N] + b0b_ref[...]
    y = jnp.maximum(_bn(t, g0b_ref[...], be0b_ref[...]), 0.0)
    o_ref[...] = jnp.dot(y, w1b_ref[...],
                         preferred_element_type=jnp.float32) + b1b_ref[...]


def kernel(x, edge_index, W0a, b0a, g0a, be0a, W1a, b1a, g_bn1, b_bn1,
           W0b, b0b, g0b, be0b, W1b, b1b):
    src = edge_index[0]
    dst = edge_index[1]
    npe = _EPAD - _E
    srcp = jnp.concatenate(
        [src, jnp.zeros((npe,), jnp.int32)]).reshape(_NW * _KCH, _CHUNK)
    dstp = jnp.concatenate(
        [dst, _N + (jnp.arange(npe, dtype=jnp.int32) % (_NPAD - _N))]
    ).reshape(_NW * _KCH, _CHUNK)
    sdm = jnp.stack([srcp, dstp], axis=1)
    zeros128 = jnp.zeros((_CHUNK, _H), jnp.float32)

    p1 = pl.pallas_call(
        _mm_k, out_shape=jax.ShapeDtypeStruct((_N, _H), jnp.float32))(x, W0a)

    part1 = _segsum(p1, sdm, zeros128).reshape(_NC, _NPAD, _H)

    z, q = pl.pallas_call(
        _stage_b_k,
        out_shape=(jax.ShapeDtypeStruct((_N, _H), jnp.float32),
                   jax.ShapeDtypeStruct((_N, _H), jnp.float32)),
    )(p1, part1, b0a.reshape(1, _H), g0a.reshape(1, _H), be0a.reshape(1, _H),
      W1a, b1a.reshape(1, _H), g_bn1.reshape(1, _H), b_bn1.reshape(1, _H),
      W0b)

    part2 = _segsum(q, sdm, zeros128).reshape(_NC, _NPAD, _H)

    out = pl.pallas_call(
        _stage_c_k,
        out_shape=jax.ShapeDtypeStruct((_N, _DOUT), jnp.float32),
    )(q, part2, b0b.reshape(1, _H), g0b.reshape(1, _H), be0b.reshape(1, _H),
      W1b, b1b.reshape(1, _DOUT))

    return (out, z)


# speedup vs baseline: 14.4635x; 1.2083x over previous
import jax
import jax.numpy as jnp
from jax import lax
from jax.experimental import pallas as pl
from jax.experimental.pallas import tpu as pltpu
from jax.experimental.pallas import tpu_sc as plsc

_N = 10000
_E = 320000
_DIN = 128
_H = 64
_DOUT = 128

_NC = 2
_NS = 16
_NW = _NC * _NS
_EW = _E // _NW
_CHUNK = 128
_KCH = _EW // _CHUNK
_TAIL = _EW - _KCH * _CHUNK

_NBUF = 4
_NBI = 8


def _segsum_body(tbl_hbm, ei_hbm, zeros_hbm, out_hbm,
                 rows, idxs, tsrc, tdst, trow, tbl_sh, acc_sh,
                 gsems, ssems, isems, psem, tsem):
    c = lax.axis_index("c")
    s = lax.axis_index("s")
    wid = s * _NC + c
    ebase = wid * _EW
    rps = _N // _NS
    r0 = s * rps

    pltpu.async_copy(zeros_hbm, acc_sh.at[pl.ds(r0, rps)], psem)
    pltpu.async_copy(tbl_hbm.at[pl.ds(r0, rps)], tbl_sh.at[pl.ds(r0, rps)],
                     psem)

    def iload_start(j, jb):
        e0 = ebase + j * _CHUNK
        pltpu.async_copy(ei_hbm.at[0, pl.ds(e0, _CHUNK)], idxs[jb].at[0],
                         isems[jb])
        pltpu.async_copy(ei_hbm.at[1, pl.ds(e0, _CHUNK)], idxs[jb].at[1],
                         isems[jb])

    def iload_wait(j, jb):
        e0 = ebase + j * _CHUNK
        pltpu.make_async_copy(ei_hbm.at[0, pl.ds(e0, _CHUNK)],
                              idxs[jb].at[0], isems[jb]).wait()
        pltpu.make_async_copy(ei_hbm.at[1, pl.ds(e0, _CHUNK)],
                              idxs[jb].at[1], isems[jb]).wait()

    def gather_start(jb, rb):
        pltpu.async_copy(tbl_sh.at[idxs[jb].at[0]], rows[rb], gsems[rb])

    def gather_wait(jb, rb):
        pltpu.make_async_copy(tbl_sh.at[idxs[jb].at[0]], rows[rb],
                              gsems[rb]).wait()

    def scat_start(jb, rb):
        pltpu.async_copy(rows[rb], acc_sh.at[idxs[jb].at[1]], ssems[rb],
                         add=True)

    def scat_wait(jb, rb):
        pltpu.make_async_copy(rows[rb], acc_sh.at[idxs[jb].at[1]],
                              ssems[rb]).wait()

    for b in range(_NBUF):
        iload_start(b, b)

    pltpu.make_async_copy(zeros_hbm, acc_sh.at[pl.ds(r0, rps)], psem).wait()
    pltpu.make_async_copy(tbl_hbm.at[pl.ds(r0, rps)],
                          tbl_sh.at[pl.ds(r0, rps)], psem).wait()
    plsc.subcore_barrier()

    def step(j, b, guard_ssem=True, do_gather=True, do_iload=True):
        rb = b % _NBUF
        gather_wait(b, rb)
        scat_start(b, rb)
        if do_gather:
            bg = (b + 2) % _NBI
            rg = (b + 2) % _NBUF
            iload_wait(j + 2, bg)
            if guard_ssem:
                scat_wait((b - 2) % _NBI, rg)
            gather_start(bg, rg)
        if do_iload:
            iload_start(j + 4, (b + 4) % _NBI)

    for b in range(2):
        iload_wait(b, b)
        gather_start(b, b)
    for b in range(_NBI):
        step(b, b, guard_ssem=(b >= 2))

    def body(g, carry):
        j0 = g * _NBI
        for b in range(_NBI):
            step(j0 + b, b)
        return carry

    lax.fori_loop(1, _KCH // _NBI, body, 0)

    for b in range(_KCH - (_KCH // _NBI) * _NBI):
        j = (_KCH // _NBI) * _NBI + b
        step(j, b, do_gather=(j + 2 < _KCH), do_iload=(j + 4 < _KCH))
    for b in range(_NBUF):
        j = _KCH - _NBUF + b
        scat_wait(j % _NBI, j % _NBUF)

    e0 = ebase + _KCH * _CHUNK
    pltpu.async_copy(ei_hbm.at[0, pl.ds(e0, _TAIL)], tsrc, tsem)
    pltpu.async_copy(ei_hbm.at[1, pl.ds(e0, _TAIL)], tdst, tsem)
    pltpu.make_async_copy(ei_hbm.at[0, pl.ds(e0, _TAIL)], tsrc, tsem).wait()
    pltpu.make_async_copy(ei_hbm.at[1, pl.ds(e0, _TAIL)], tdst, tsem).wait()
    pltpu.async_copy(tbl_sh.at[tsrc], trow, tsem)
    pltpu.make_async_copy(tbl_sh.at[tsrc], trow, tsem).wait()
    pltpu.sync_copy(trow, acc_sh.at[tdst], add=True)

    plsc.subcore_barrier()
    pltpu.sync_copy(acc_sh.at[pl.ds(r0, rps)],
                    out_hbm.at[pl.ds(c * _N + r0, rps)])


_segsum = pl.kernel(
    _segsum_body,
    mesh=plsc.VectorSubcoreMesh(core_axis_name="c", subcore_axis_name="s"),
    out_type=jax.ShapeDtypeStruct((_NC * _N, _H), jnp.float32),
    scratch_types=[
        tuple(pltpu.VMEM((_CHUNK, _H), jnp.float32) for _ in range(_NBUF)),
        tuple(pltpu.VMEM((2, _CHUNK), jnp.int32) for _ in range(_NBI)),
        pltpu.VMEM((_TAIL,), jnp.int32),
        pltpu.VMEM((_TAIL,), jnp.int32),
        pltpu.VMEM((_TAIL, _H), jnp.float32),
        pltpu.VMEM_SHARED((_N, _H), jnp.float32),
        pltpu.VMEM_SHARED((_N, _H), jnp.float32),
        tuple(pltpu.SemaphoreType.DMA for _ in range(_NBUF)),
        tuple(pltpu.SemaphoreType.DMA for _ in range(_NBUF)),
        tuple(pltpu.SemaphoreType.DMA for _ in range(_NBI)),
        pltpu.SemaphoreType.DMA,
        pltpu.SemaphoreType.DMA,
    ],
    compiler_params=pltpu.CompilerParams(use_tc_tiling_on_sc=False),
)


def _bn(t, g, b):
    mean = jnp.mean(t, axis=0, keepdims=True)
    var = jnp.mean(jnp.square(t - mean), axis=0, keepdims=True)
    return (t - mean) * lax.rsqrt(var + 1e-5) * g + b


def _mm_k(x_ref, w_ref, o_ref):
    o_ref[...] = jnp.dot(x_ref[...], w_ref[...],
                         preferred_element_type=jnp.float32)


def _stage_b_k(p_ref, part_ref, b0a_ref, g0a_ref, be0a_ref, w1a_ref, b1a_ref,
               gbn_ref, bbn_ref, w0b_ref, z_ref, q_ref):
    agg = part_ref[0, :, :] + part_ref[1, :, :]
    t = p_ref[...] + agg + b0a_ref[...]
    y = jnp.maximum(_bn(t, g0a_ref[...], be0a_ref[...]), 0.0)
    z = jnp.dot(y, w1a_ref[...], preferred_element_type=jnp.float32) + b1a_ref[...]
    z_ref[...] = z
    hh = jnp.maximum(_bn(z, gbn_ref[...], bbn_ref[...]), 0.0)
    q_ref[...] = jnp.dot(hh, w0b_ref[...], preferred_element_type=jnp.float32)


def _stage_c_k(q_ref, part_ref, b0b_ref, g0b_ref, be0b_ref, w1b_ref, b1b_ref,
               o_ref):
    agg = part_ref[0, :, :] + part_ref[1, :, :]
    t = q_ref[...] + agg + b0b_ref[...]
    y = jnp.maximum(_bn(t, g0b_ref[...], be0b_ref[...]), 0.0)
    o_ref[...] = jnp.dot(y, w1b_ref[...],
                         preferred_element_type=jnp.float32) + b1b_ref[...]


def kernel(x, edge_index, W0a, b0a, g0a, be0a, W1a, b1a, g_bn1, b_bn1,
           W0b, b0b, g0b, be0b, W1b, b1b):
    zeros_sub = jnp.zeros((_N // _NS, _H), jnp.float32)

    p1 = pl.pallas_call(
        _mm_k, out_shape=jax.ShapeDtypeStruct((_N, _H), jnp.float32))(x, W0a)

    part1 = _segsum(p1, edge_index, zeros_sub).reshape(_NC, _N, _H)

    z, q = pl.pallas_call(
        _stage_b_k,
        out_shape=(jax.ShapeDtypeStruct((_N, _H), jnp.float32),
                   jax.ShapeDtypeStruct((_N, _H), jnp.float32)),
    )(p1, part1, b0a.reshape(1, _H), g0a.reshape(1, _H), be0a.reshape(1, _H),
      W1a, b1a.reshape(1, _H), g_bn1.reshape(1, _H), b_bn1.reshape(1, _H),
      W0b)

    part2 = _segsum(q, edge_index, zeros_sub).reshape(_NC, _N, _H)

    out = pl.pallas_call(
        _stage_c_k,
        out_shape=jax.ShapeDtypeStruct((_N, _DOUT), jnp.float32),
    )(q, part2, b0b.reshape(1, _H), g0b.reshape(1, _H), be0b.reshape(1, _H),
      W1b, b1b.reshape(1, _DOUT))

    return (out, z)
